# Initial kernel scaffold; baseline (speedup 1.0000x reference)
import jax
import jax.numpy as jnp
from jax.experimental import pallas as pl


def kernel(input_ids, prompt_ids, embed_weight):
    raise NotImplementedError("write your pallas kernel here")


# speedup vs baseline: 1.8395x; 1.8395x over previous
import functools

import jax
import jax.numpy as jnp
from jax.experimental import pallas as pl
from jax.experimental.pallas import tpu as pltpu

B = 16
L = 2048
P = 2048
D = 1024
PBLK = 512
NBLK = P // PBLK


def _kernel_body(pids_ref, inp_ref, w_ref, ids_out_ref, emb_out_ref, scratch_ref):
    b = pl.program_id(0)

    @pl.when(b == 0)
    def _gather():
        for j in range(NBLK):
            base = j * PBLK
            idsblk = pids_ref[0:1, base:base + PBLK]
            rows = jax.lax.broadcasted_iota(jnp.int32, (P, PBLK), 0)
            onehot_t = (rows == idsblk).astype(jnp.float32)
            g = jax.lax.dot_general(
                onehot_t, w_ref[...], (((0,), (0,)), ((), ())),
                preferred_element_type=jnp.float32,
                precision=jax.lax.Precision.HIGHEST,
            )
            hit = jax.lax.dot_general(
                onehot_t, jnp.ones((P, 1), jnp.float32), (((0,), (0,)), ((), ())),
                preferred_element_type=jnp.float32,
                precision=jax.lax.Precision.HIGHEST,
            )
            g = jnp.where(hit > 0.5, g, jnp.float32(jnp.nan))
            scratch_ref[base:base + PBLK, :] = g

    emb_out_ref[0] = scratch_ref[...]
    ids_out_ref[0, 0, pl.ds(0, P)] = pids_ref[0]
    ids_out_ref[0, 0, pl.ds(P, L)] = inp_ref[0, 0]


@functools.partial(jax.jit)
def kernel(input_ids, prompt_ids, embed_weight):
    pids2 = prompt_ids.reshape(1, P)
    inp3 = input_ids.reshape(B, 1, L)
    ids_out, emb_out = pl.pallas_call(
        _kernel_body,
        grid=(B,),
        in_specs=[
            pl.BlockSpec((1, P), lambda b: (0, 0)),
            pl.BlockSpec((1, 1, L), lambda b: (b, 0, 0)),
            pl.BlockSpec((P, D), lambda b: (0, 0)),
        ],
        out_specs=[
            pl.BlockSpec((1, 1, P + L), lambda b: (b, 0, 0)),
            pl.BlockSpec((1, P, D), lambda b: (b, 0, 0)),
        ],
        out_shape=[
            jax.ShapeDtypeStruct((B, 1, P + L), jnp.int32),
            jax.ShapeDtypeStruct((B, P, D), jnp.float32),
        ],
        scratch_shapes=[pltpu.VMEM((P, D), jnp.float32)],
    )(pids2, inp3, embed_weight)
    return ids_out.reshape(B, P + L), emb_out


# speedup vs baseline: 2.2237x; 1.2088x over previous
import functools

import jax
import jax.numpy as jnp
from jax.experimental import pallas as pl
from jax.experimental.pallas import tpu as pltpu

B = 16
L = 2048
P = 2048
D = 1024
PBLK = 512
NBLK = P // PBLK


def _kernel_body(pids_ref, pblk_ref, inp_ref, w_ref, ids_out_ref, emb_out_ref,
                 scratch_ref):
    b = pl.program_id(1)

    @pl.when(b == 0)
    def _gather():
        idsblk = pblk_ref[...]
        rows = jax.lax.broadcasted_iota(jnp.int32, (P, PBLK), 0)
        onehot_t = (rows == idsblk).astype(jnp.float32)
        g = jax.lax.dot_general(
            onehot_t, w_ref[...], (((0,), (0,)), ((), ())),
            preferred_element_type=jnp.float32,
            precision=jax.lax.Precision.HIGHEST,
        )
        hit = jax.lax.dot_general(
            onehot_t, jnp.ones((P, 1), jnp.float32), (((0,), (0,)), ((), ())),
            preferred_element_type=jnp.float32,
        )
        scratch_ref[...] = jnp.where(hit > 0.5, g, jnp.float32(jnp.nan))

    emb_out_ref[0] = scratch_ref[...]
    ids_out_ref[0, 0, pl.ds(0, P)] = pids_ref[0]
    ids_out_ref[0, 0, pl.ds(P, L)] = inp_ref[0, 0]


@functools.partial(jax.jit)
def kernel(input_ids, prompt_ids, embed_weight):
    pids2 = prompt_ids.reshape(1, P)
    inp3 = input_ids.reshape(B, 1, L)
    ids_out, emb_out = pl.pallas_call(
        _kernel_body,
        grid=(NBLK, B),
        in_specs=[
            pl.BlockSpec((1, P), lambda j, b: (0, 0)),
            pl.BlockSpec((1, PBLK), lambda j, b: (0, j)),
            pl.BlockSpec((1, 1, L), lambda j, b: (b, 0, 0)),
            pl.BlockSpec((P, D), lambda j, b: (0, 0)),
        ],
        out_specs=[
            pl.BlockSpec((1, 1, P + L), lambda j, b: (b, 0, 0)),
            pl.BlockSpec((1, PBLK, D), lambda j, b: (b, j, 0)),
        ],
        out_shape=[
            jax.ShapeDtypeStruct((B, 1, P + L), jnp.int32),
            jax.ShapeDtypeStruct((B, PBLK * NBLK, D), jnp.float32),
        ],
        scratch_shapes=[pltpu.VMEM((PBLK, D), jnp.float32)],
    )(pids2, pids2, inp3, embed_weight)
    return ids_out.reshape(B, P + L), emb_out
